# Initial kernel scaffold; baseline (speedup 1.0000x reference)
import jax
import jax.numpy as jnp
from jax.experimental import pallas as pl


def kernel(x, table):
    raise NotImplementedError("write your pallas kernel here")


# speedup vs baseline: 1.3532x; 1.3532x over previous
import jax
import jax.numpy as jnp
from jax import lax
from jax.experimental import pallas as pl
from jax.experimental.pallas import tpu as pltpu
from jax.experimental.pallas import tpu_sc as plsc

BATCH = 16384
HIST_LEN = 50
EMBED_DIM = 64
PAD_DIM = 128
NUM_IDX = BATCH * HIST_LEN
NC, NS = 2, 16
NW = NC * NS
B_PER_W = NUM_IDX // NW
WIN = 128
STEPS = B_PER_W // WIN


def kernel(x, table):
    idx = x.reshape(NUM_IDX).astype(jnp.int32)
    table128 = jnp.pad(table, ((0, 0), (0, PAD_DIM - EMBED_DIM)))
    mesh = plsc.VectorSubcoreMesh(core_axis_name="c", subcore_axis_name="s")

    @jax.jit
    @pl.kernel(
        out_type=jax.ShapeDtypeStruct((NUM_IDX, EMBED_DIM), table.dtype),
        mesh=mesh,
        scratch_types=[
            pltpu.VMEM((WIN,), jnp.int32),
            pltpu.VMEM((WIN, PAD_DIM), jnp.float32),
            pltpu.VMEM((WIN, EMBED_DIM), jnp.float32),
            pltpu.SemaphoreType.DMA,
        ],
    )
    def gather_kernel(table_hbm, idx_hbm, out_hbm, idx_v, rows_v, out_c, sem):
        wid = lax.axis_index("s") * NC + lax.axis_index("c")
        base = wid * B_PER_W

        @pl.loop(0, STEPS)
        def _(g):
            off = base + g * WIN
            pltpu.sync_copy(idx_hbm.at[pl.ds(off, WIN)], idx_v)
            pltpu.async_copy(table_hbm.at[idx_v], rows_v, sem).wait()

            @pl.loop(0, WIN)
            def _(r):
                for c in range(EMBED_DIM // 16):
                    slc = (pl.ds(r, 1), pl.ds(16 * c, 16))
                    out_c.at[*slc][...] = rows_v.at[*slc][...]

            pltpu.sync_copy(out_c, out_hbm.at[pl.ds(off, WIN)])

    out = gather_kernel(table128, idx)
    return out.reshape(BATCH, HIST_LEN, EMBED_DIM)


# speedup vs baseline: 1.9834x; 1.4658x over previous
import jax
import jax.numpy as jnp
from jax import lax
from jax.experimental import pallas as pl
from jax.experimental.pallas import tpu as pltpu
from jax.experimental.pallas import tpu_sc as plsc

BATCH = 16384
HIST = 50
DIM = 64
PAD = 128
NC, NS = 2, 16
NW = NC * NS
B_PER_W = BATCH // NW
G = 4
STEPS = B_PER_W // G
IDX_PER_STEP = G * HIST
SPLITS = ((0, 128), (128, 72))


def kernel(x, table):
    idx = x.reshape(BATCH * HIST).astype(jnp.int32)
    table128 = jnp.pad(table, ((0, 0), (0, PAD - DIM)))
    mesh = plsc.VectorSubcoreMesh(core_axis_name="c", subcore_axis_name="s")

    @jax.jit
    @pl.kernel(
        out_type=jax.ShapeDtypeStruct((BATCH, HIST, DIM), table.dtype),
        mesh=mesh,
        scratch_types=[
            pltpu.VMEM((IDX_PER_STEP,), jnp.int32),
            pltpu.VMEM((IDX_PER_STEP,), jnp.int32),
            pltpu.VMEM((IDX_PER_STEP, PAD), jnp.float32),
            pltpu.VMEM((IDX_PER_STEP, PAD), jnp.float32),
            pltpu.VMEM((IDX_PER_STEP, DIM), jnp.float32),
            pltpu.SemaphoreType.DMA,
            pltpu.SemaphoreType.DMA,
        ],
    )
    def gk(table_hbm, idx_hbm, out_hbm, idx0, idx1, rows0, rows1, out_c, sg0, sg1):
        wid = lax.axis_index("s") * NC + lax.axis_index("c")
        base_b = wid * B_PER_W

        def fire(s, idxr, rowsr, sem):
            off = (base_b + s * G) * HIST
            pltpu.sync_copy(idx_hbm.at[pl.ds(off, IDX_PER_STEP)], idxr)
            for o, n in SPLITS:
                pltpu.async_copy(
                    table_hbm.at[idxr.at[pl.ds(o, n)]],
                    rowsr.at[pl.ds(o, n)],
                    sem,
                )

        def wait_gather(idxr, rowsr, sem):
            for o, n in SPLITS:
                pltpu.make_async_copy(
                    table_hbm.at[idxr.at[pl.ds(o, n)]],
                    rowsr.at[pl.ds(o, n)],
                    sem,
                ).wait()

        def body(s, idxr, rowsr, sem):
            wait_gather(idxr, rowsr, sem)

            @pl.loop(0, IDX_PER_STEP)
            def _(r):
                for c in range(DIM // 16):
                    slc = (pl.ds(r, 1), pl.ds(16 * c, 16))
                    out_c.at[*slc][...] = rowsr.at[*slc][...]

            nxt = s + 2

            @pl.when(nxt < STEPS)
            def _():
                fire(nxt, idxr, rowsr, sem)

            b = base_b + s * G
            for j in range(G):
                pltpu.sync_copy(
                    out_c.at[pl.ds(j * HIST, HIST)], out_hbm.at[b + j]
                )

        fire(0, idx0, rows0, sg0)
        fire(1, idx1, rows1, sg1)

        @pl.loop(0, STEPS, step=2)
        def _(g):
            body(g, idx0, rows0, sg0)
            body(g + 1, idx1, rows1, sg1)

    out = gk(table128, idx)
    return out
